# baseline (speedup 1.0000x reference)
import jax
import jax.numpy as jnp
from jax.experimental import pallas as pl


S, M, E = 2048, 4096, 16
C = 2 * S // E


def _matmul_body(x_ref, w_ref, out_ref):
    out_ref[...] = jax.lax.dot_general(
        x_ref[...], w_ref[...],
        dimension_numbers=(((1,), (1,)), ((), ())),
        preferred_element_type=jnp.float32,
    )


def _logits_tc(x, W):
    return pl.pallas_call(
        _matmul_body,
        grid=(8,),
        in_specs=[
            pl.BlockSpec((S // 8, M), lambda i: (i, 0)),
            pl.BlockSpec((E, M), lambda i: (0, 0)),
        ],
        out_specs=pl.BlockSpec((S // 8, E), lambda i: (i, 0)),
        out_shape=jax.ShapeDtypeStruct((S, E), jnp.float32),
    )(x, W)


def _gating_jnp(logits):
    gates = jax.nn.softmax(logits.astype(jnp.float32), axis=1)
    indices1_s = jnp.argmax(gates, axis=1)
    mask1 = jax.nn.one_hot(indices1_s, E, dtype=jnp.int32)
    logits_except1 = jnp.where(mask1.astype(bool), -jnp.inf, logits)
    indices2_s = jnp.argmax(logits_except1, axis=1)
    mask2 = jax.nn.one_hot(indices2_s, E, dtype=jnp.int32)
    locations1 = jnp.cumsum(mask1, axis=0) - 1
    locations2 = jnp.cumsum(mask2, axis=0) - 1
    locations2 = locations2 + jnp.sum(mask1, axis=0, keepdims=True)
    me = jnp.mean(gates, axis=0)
    ce = jnp.mean(mask1.astype(jnp.float32), axis=0)
    l_aux = jnp.mean(me * ce)
    mask1 = mask1 * (locations1 < C).astype(jnp.int32)
    mask2 = mask2 * (locations2 < C).astype(jnp.int32)
    gates1_s = jnp.sum(gates * mask1.astype(gates.dtype), axis=1)
    gates2_s = jnp.sum(gates * mask2.astype(gates.dtype), axis=1)
    denom_s = jnp.maximum(gates1_s + gates2_s, jnp.finfo(jnp.float32).eps)
    gates1_s = gates1_s / denom_s
    gates2_s = gates2_s / denom_s
    locations1_s = jnp.sum(locations1 * mask1, axis=1)
    locations2_s = jnp.sum(locations2 * mask2, axis=1)
    combine1_sec = gates1_s[:, None, None] * jax.nn.one_hot(locations1_s, C, dtype=jnp.float32)[:, None, :]
    combine2_sec = gates2_s[:, None, None] * jax.nn.one_hot(locations2_s, C, dtype=jnp.float32)[:, None, :]
    combine_weights = combine1_sec + combine2_sec
    dispatch_mask = combine_weights.astype(bool)
    return l_aux.astype(logits.dtype), combine_weights.astype(logits.dtype), dispatch_mask


def kernel(input, W):
    logits = _logits_tc(input, W)
    return _gating_jnp(logits)
